# baseline (speedup 1.0000x reference)
import functools

import jax
import jax.numpy as jnp
from jax import lax
from jax.experimental import pallas as pl
from jax.experimental.pallas import tpu as pltpu
from jax.experimental.pallas import tpu_sc as plsc

N_NODES = 10000
N_EDGES = 320000
HIDDEN = 128
NUM_RBF = 16
CUTOFF_UPPER = 5.0

NC = 2
NS = 16
L = 16
NW = NC * NS
EPW = N_EDGES // NW
CH = 80
NCHW = EPW // CH

EB = 2560
RB = 1000

_HI = jax.lax.Precision.DEFAULT


def _silu(v):
    return v * jax.nn.sigmoid(v)


def _proj_body(x_ref, w_ref, o_ref):
    o_ref[:] = jnp.dot(x_ref[:], w_ref[:], precision=_HI,
                       preferred_element_type=jnp.float32)


def _proj(x, lin1_w):
    return pl.pallas_call(
        _proj_body,
        grid=(N_NODES // RB,),
        in_specs=[pl.BlockSpec((RB, HIDDEN), lambda i: (i, 0)),
                  pl.BlockSpec((HIDDEN, HIDDEN), lambda i: (0, 0))],
        out_specs=pl.BlockSpec((RB, HIDDEN), lambda i: (i, 0)),
        out_shape=jax.ShapeDtypeStruct((N_NODES, HIDDEN), jnp.float32),
    )(x, lin1_w.T)


def _cutoff_body(ew_ref, o_ref):
    d = ew_ref[:]
    c = 0.5 * (jnp.cos(d * (jnp.pi / CUTOFF_UPPER)) + 1.0)
    o_ref[:] = c * (d < CUTOFF_UPPER).astype(jnp.float32)


def _cutoff(edge_weight):
    ew2 = edge_weight.reshape(N_EDGES // HIDDEN, HIDDEN)
    c2 = pl.pallas_call(
        _cutoff_body,
        out_shape=jax.ShapeDtypeStruct(ew2.shape, jnp.float32),
    )(ew2)
    return c2.reshape(N_EDGES, 1)


def _filter_body(attr_ref, c_ref, w1t_ref, b1_ref, w2t_ref, b2_ref, o_ref):
    h = jnp.dot(attr_ref[:], w1t_ref[:], precision=_HI,
                preferred_element_type=jnp.float32) + b1_ref[:]
    h = _silu(h)
    w = jnp.dot(h, w2t_ref[:], precision=_HI,
                preferred_element_type=jnp.float32) + b2_ref[:]
    o_ref[:] = w * c_ref[:]


def _filter(edge_attr, cutoff_col, mlp_w1, mlp_b1, mlp_w2, mlp_b2):
    return pl.pallas_call(
        _filter_body,
        grid=(N_EDGES // EB,),
        in_specs=[
            pl.BlockSpec((EB, NUM_RBF), lambda i: (i, 0)),
            pl.BlockSpec((EB, 1), lambda i: (i, 0)),
            pl.BlockSpec((NUM_RBF, HIDDEN), lambda i: (0, 0)),
            pl.BlockSpec((1, HIDDEN), lambda i: (0, 0)),
            pl.BlockSpec((HIDDEN, HIDDEN), lambda i: (0, 0)),
            pl.BlockSpec((1, HIDDEN), lambda i: (0, 0)),
        ],
        out_specs=pl.BlockSpec((EB, HIDDEN), lambda i: (i, 0)),
        out_shape=jax.ShapeDtypeStruct((N_EDGES, HIDDEN), jnp.float32),
    )(edge_attr, cutoff_col, mlp_w1.T,
      mlp_b1.reshape(1, HIDDEN), mlp_w2.T, mlp_b2.reshape(1, HIDDEN))


def _sc_body(xl_hbm, w_hbm, src_hbm, dst_hbm, zeros_hbm, out_hbm,
             srci0, srci1, dsti0, dsti1, xlg0, xlg1, wv, agg_sh,
             gsem0, gsem1, ssem0, ssem1):
    c = lax.axis_index("c")
    s = lax.axis_index("s")
    w_id = c * NS + s

    RBLK = 400
    NRB = N_NODES // RBLK

    @pl.loop(0, 2)
    def _(kk):
        b = s + kk * NS

        @pl.when(b < NRB)
        def _():
            pltpu.sync_copy(zeros_hbm.at[pl.ds(b * RBLK, RBLK)],
                            agg_sh.at[pl.ds(b * RBLK, RBLK)])

    plsc.subcore_barrier()

    def load_idx(j, srci, dsti):
        off = w_id * EPW + j * CH
        pltpu.sync_copy(src_hbm.at[pl.ds(off, CH)], srci)
        pltpu.sync_copy(dst_hbm.at[pl.ds(off, CH)], dsti)

    def start_gather(srci, xlg, gsem):
        pltpu.async_copy(xl_hbm.at[srci], xlg, gsem)

    def wait_scatter(xlg, dsti, ssem):
        pltpu.make_async_copy(xlg, agg_sh.at[dsti], ssem).wait()

    def process(j, xlg, srci, dsti, gsem, ssem):
        pltpu.make_async_copy(xl_hbm.at[srci], xlg, gsem).wait()
        pltpu.sync_copy(w_hbm.at[pl.ds(w_id * EPW + j * CH, CH)], wv)

        @pl.loop(0, CH, step=4)
        def _(r0):
            for dr in range(4):
                r = r0 + dr
                for j2 in range(HIDDEN // L):
                    sl = pl.ds(j2 * L, L)
                    xlg[r, sl] = xlg[r, sl] * wv[r, sl]

        pltpu.async_copy(xlg, agg_sh.at[dsti], ssem, add=True)

    load_idx(0, srci0, dsti0)
    start_gather(srci0, xlg0, gsem0)

    @pl.loop(0, (NCHW + 1) // 2)
    def _(t):
        j = t * 2

        @pl.when(j + 1 < NCHW)
        def _():
            @pl.when(j - 1 >= 0)
            def _():
                wait_scatter(xlg1, dsti1, ssem1)

            load_idx(j + 1, srci1, dsti1)
            start_gather(srci1, xlg1, gsem1)

        process(j, xlg0, srci0, dsti0, gsem0, ssem0)

        @pl.when(j + 2 < NCHW)
        def _():
            wait_scatter(xlg0, dsti0, ssem0)
            load_idx(j + 2, srci0, dsti0)
            start_gather(srci0, xlg0, gsem0)

        @pl.when(j + 1 < NCHW)
        def _():
            process(j + 1, xlg1, srci1, dsti1, gsem1, ssem1)

    wait_scatter(xlg0, dsti0, ssem0)
    wait_scatter(xlg1, dsti1, ssem1)

    plsc.subcore_barrier()

    @pl.loop(0, 2)
    def _(kk):
        b = s + kk * NS

        @pl.when(b < NRB)
        def _():
            pltpu.sync_copy(agg_sh.at[pl.ds(b * RBLK, RBLK)],
                            out_hbm.at[pl.ds(c * N_NODES + b * RBLK, RBLK)])


def _sc_conv(xl, w_edges, src, dst, zeros):
    mesh = plsc.VectorSubcoreMesh(core_axis_name="c", subcore_axis_name="s")
    k = pl.kernel(
        _sc_body,
        out_type=jax.ShapeDtypeStruct((NC * N_NODES, HIDDEN), jnp.float32),
        mesh=mesh,
        scratch_types=[
            pltpu.VMEM((CH,), jnp.int32),
            pltpu.VMEM((CH,), jnp.int32),
            pltpu.VMEM((CH,), jnp.int32),
            pltpu.VMEM((CH,), jnp.int32),
            pltpu.VMEM((CH, HIDDEN), jnp.float32),
            pltpu.VMEM((CH, HIDDEN), jnp.float32),
            pltpu.VMEM((CH, HIDDEN), jnp.float32),
            pltpu.VMEM_SHARED((N_NODES, HIDDEN), jnp.float32),
            pltpu.SemaphoreType.DMA,
            pltpu.SemaphoreType.DMA,
            pltpu.SemaphoreType.DMA,
            pltpu.SemaphoreType.DMA,
        ],
    )
    return k(xl, w_edges, src, dst, zeros)


def _tail_body(nat_ref, a0_ref, a1_ref, w2t_ref, b2_ref, wt_ref, b_ref, o_ref):
    agg = a0_ref[:] + a1_ref[:]
    i = pl.program_id(0)
    rows = i * RB + lax.broadcasted_iota(jnp.int32, (RB, 1), 0)
    agg = jnp.where(rows < nat_ref[0], agg, 0.0)
    o = jnp.dot(agg, w2t_ref[:], precision=_HI,
                preferred_element_type=jnp.float32) + b2_ref[:]
    o = _silu(o)
    o_ref[:] = jnp.dot(o, wt_ref[:], precision=_HI,
                       preferred_element_type=jnp.float32) + b_ref[:]


def _tail(aggp, n_atoms, lin2_w, lin2_b, lin_w, lin_b):
    nat = jnp.asarray(n_atoms, jnp.int32).reshape(1)
    return pl.pallas_call(
        _tail_body,
        grid=(N_NODES // RB,),
        in_specs=[
            pl.BlockSpec(memory_space=pltpu.SMEM),
            pl.BlockSpec((RB, HIDDEN), lambda i: (i, 0)),
            pl.BlockSpec((RB, HIDDEN), lambda i: (i + N_NODES // RB, 0)),
            pl.BlockSpec((HIDDEN, HIDDEN), lambda i: (0, 0)),
            pl.BlockSpec((1, HIDDEN), lambda i: (0, 0)),
            pl.BlockSpec((HIDDEN, HIDDEN), lambda i: (0, 0)),
            pl.BlockSpec((1, HIDDEN), lambda i: (0, 0)),
        ],
        out_specs=pl.BlockSpec((RB, HIDDEN), lambda i: (i, 0)),
        out_shape=jax.ShapeDtypeStruct((N_NODES, HIDDEN), jnp.float32),
    )(nat, aggp, aggp, lin2_w.T, lin2_b.reshape(1, HIDDEN), lin_w.T,
      lin_b.reshape(1, HIDDEN))


def kernel(x, edge_index, edge_weight, edge_attr, n_atoms,
           mlp_w1, mlp_b1, mlp_w2, mlp_b2,
           lin1_w, lin2_w, lin2_b, lin_w, lin_b):
    src = edge_index[1].astype(jnp.int32)
    dst = edge_index[0].astype(jnp.int32)
    xl = _proj(x, lin1_w)
    cutoff_col = _cutoff(edge_weight)
    w_edges = _filter(edge_attr, cutoff_col, mlp_w1, mlp_b1, mlp_w2, mlp_b2)
    zeros = jnp.zeros((N_NODES, HIDDEN), jnp.float32)
    aggp = _sc_conv(xl, w_edges, src, dst, zeros)
    return _tail(aggp, n_atoms, lin2_w, lin2_b, lin_w, lin_b)


# speedup vs baseline: 1.0522x; 1.0522x over previous
import functools

import jax
import jax.numpy as jnp
from jax import lax
from jax.experimental import pallas as pl
from jax.experimental.pallas import tpu as pltpu
from jax.experimental.pallas import tpu_sc as plsc

N_NODES = 10000
N_EDGES = 320000
HIDDEN = 128
NUM_RBF = 16
CUTOFF_UPPER = 5.0

NC = 2
NS = 16
L = 16
NW = NC * NS
EPW = N_EDGES // NW
CH = 80
NCHW = EPW // CH

EB = 2560
RB = 1000

_HI = jax.lax.Precision.DEFAULT


def _silu(v):
    return v * jax.nn.sigmoid(v)


def _proj_body(x_ref, w_ref, o_ref):
    o_ref[:] = jnp.dot(x_ref[:], w_ref[:], precision=_HI,
                       preferred_element_type=jnp.float32)


def _proj(x, lin1_w):
    return pl.pallas_call(
        _proj_body,
        grid=(N_NODES // RB,),
        in_specs=[pl.BlockSpec((RB, HIDDEN), lambda i: (i, 0)),
                  pl.BlockSpec((HIDDEN, HIDDEN), lambda i: (0, 0))],
        out_specs=pl.BlockSpec((RB, HIDDEN), lambda i: (i, 0)),
        out_shape=jax.ShapeDtypeStruct((N_NODES, HIDDEN), jnp.float32),
    )(x, lin1_w.T)


def _cutoff_body(ew_ref, o_ref):
    d = ew_ref[:]
    c = 0.5 * (jnp.cos(d * (jnp.pi / CUTOFF_UPPER)) + 1.0)
    o_ref[:] = c * (d < CUTOFF_UPPER).astype(jnp.float32)


def _cutoff(edge_weight):
    ew2 = edge_weight.reshape(N_EDGES // HIDDEN, HIDDEN)
    c2 = pl.pallas_call(
        _cutoff_body,
        out_shape=jax.ShapeDtypeStruct(ew2.shape, jnp.float32),
    )(ew2)
    return c2.reshape(N_EDGES, 1)


def _filter_body(attr_ref, c_ref, w1t_ref, b1_ref, w2t_ref, b2_ref, o_ref):
    h = jnp.dot(attr_ref[:], w1t_ref[:], precision=_HI,
                preferred_element_type=jnp.float32) + b1_ref[:]
    h = _silu(h)
    w = jnp.dot(h, w2t_ref[:], precision=_HI,
                preferred_element_type=jnp.float32) + b2_ref[:]
    o_ref[:] = w * c_ref[:]


def _filter(edge_attr, cutoff_col, mlp_w1, mlp_b1, mlp_w2, mlp_b2):
    return pl.pallas_call(
        _filter_body,
        grid=(N_EDGES // EB,),
        in_specs=[
            pl.BlockSpec((EB, NUM_RBF), lambda i: (i, 0)),
            pl.BlockSpec((EB, 1), lambda i: (i, 0)),
            pl.BlockSpec((NUM_RBF, HIDDEN), lambda i: (0, 0)),
            pl.BlockSpec((1, HIDDEN), lambda i: (0, 0)),
            pl.BlockSpec((HIDDEN, HIDDEN), lambda i: (0, 0)),
            pl.BlockSpec((1, HIDDEN), lambda i: (0, 0)),
        ],
        out_specs=pl.BlockSpec((EB, HIDDEN), lambda i: (i, 0)),
        out_shape=jax.ShapeDtypeStruct((N_EDGES, HIDDEN), jnp.float32),
    )(edge_attr, cutoff_col, mlp_w1.T,
      mlp_b1.reshape(1, HIDDEN), mlp_w2.T, mlp_b2.reshape(1, HIDDEN))


def _sc_body(xl_hbm, w_hbm, idx_hbm, zeros_hbm, out_hbm,
             idx0, idx1, xlg0, xlg1, wv, agg_sh,
             gsem0, gsem1, ssem0, ssem1):
    c = lax.axis_index("c")
    s = lax.axis_index("s")
    w_id = c * NS + s

    RBLK = 400
    NRB = N_NODES // RBLK

    @pl.loop(0, 2)
    def _(kk):
        b = s + kk * NS

        @pl.when(b < NRB)
        def _():
            pltpu.sync_copy(zeros_hbm.at[pl.ds(b * RBLK, RBLK)],
                            agg_sh.at[pl.ds(b * RBLK, RBLK)])

    plsc.subcore_barrier()

    def load_idx(j, idxb):
        pltpu.sync_copy(idx_hbm.at[w_id * NCHW + j], idxb)

    def start_gather(idxb, xlg, gsem):
        pltpu.async_copy(xl_hbm.at[idxb.at[0]], xlg, gsem)

    def wait_scatter(xlg, idxb, ssem):
        pltpu.make_async_copy(xlg, agg_sh.at[idxb.at[1]], ssem).wait()

    def process(j, xlg, idxb, gsem, ssem):
        pltpu.make_async_copy(xl_hbm.at[idxb.at[0]], xlg, gsem).wait()
        pltpu.sync_copy(w_hbm.at[pl.ds(w_id * EPW + j * CH, CH)], wv)

        @pl.loop(0, CH, step=4)
        def _(r0):
            for dr in range(4):
                r = r0 + dr
                for j2 in range(HIDDEN // L):
                    sl = pl.ds(j2 * L, L)
                    xlg[r, sl] = xlg[r, sl] * wv[r, sl]

        pltpu.async_copy(xlg, agg_sh.at[idxb.at[1]], ssem, add=True)

    load_idx(0, idx0)
    start_gather(idx0, xlg0, gsem0)

    @pl.loop(0, (NCHW + 1) // 2)
    def _(t):
        j = t * 2

        @pl.when(j + 1 < NCHW)
        def _():
            @pl.when(j - 1 >= 0)
            def _():
                wait_scatter(xlg1, idx1, ssem1)

            load_idx(j + 1, idx1)
            start_gather(idx1, xlg1, gsem1)

        process(j, xlg0, idx0, gsem0, ssem0)

        @pl.when(j + 2 < NCHW)
        def _():
            wait_scatter(xlg0, idx0, ssem0)
            load_idx(j + 2, idx0)
            start_gather(idx0, xlg0, gsem0)

        @pl.when(j + 1 < NCHW)
        def _():
            process(j + 1, xlg1, idx1, gsem1, ssem1)

    wait_scatter(xlg0, idx0, ssem0)
    wait_scatter(xlg1, idx1, ssem1)

    plsc.subcore_barrier()

    @pl.loop(0, 2)
    def _(kk):
        b = s + kk * NS

        @pl.when(b < NRB)
        def _():
            pltpu.sync_copy(agg_sh.at[pl.ds(b * RBLK, RBLK)],
                            out_hbm.at[pl.ds(c * N_NODES + b * RBLK, RBLK)])


def _sc_conv(xl, w_edges, src, dst, zeros):
    idx2 = jnp.stack([src.reshape(NW * NCHW, CH),
                      dst.reshape(NW * NCHW, CH)], axis=1)
    mesh = plsc.VectorSubcoreMesh(core_axis_name="c", subcore_axis_name="s")
    k = pl.kernel(
        _sc_body,
        out_type=jax.ShapeDtypeStruct((NC * N_NODES, HIDDEN), jnp.float32),
        mesh=mesh,
        scratch_types=[
            pltpu.VMEM((2, CH), jnp.int32),
            pltpu.VMEM((2, CH), jnp.int32),
            pltpu.VMEM((CH, HIDDEN), jnp.float32),
            pltpu.VMEM((CH, HIDDEN), jnp.float32),
            pltpu.VMEM((CH, HIDDEN), jnp.float32),
            pltpu.VMEM_SHARED((N_NODES, HIDDEN), jnp.float32),
            pltpu.SemaphoreType.DMA,
            pltpu.SemaphoreType.DMA,
            pltpu.SemaphoreType.DMA,
            pltpu.SemaphoreType.DMA,
        ],
    )
    return k(xl, w_edges, idx2, zeros)


def _tail_body(nat_ref, a0_ref, a1_ref, w2t_ref, b2_ref, wt_ref, b_ref, o_ref):
    agg = a0_ref[:] + a1_ref[:]
    i = pl.program_id(0)
    rows = i * RB + lax.broadcasted_iota(jnp.int32, (RB, 1), 0)
    agg = jnp.where(rows < nat_ref[0], agg, 0.0)
    o = jnp.dot(agg, w2t_ref[:], precision=_HI,
                preferred_element_type=jnp.float32) + b2_ref[:]
    o = _silu(o)
    o_ref[:] = jnp.dot(o, wt_ref[:], precision=_HI,
                       preferred_element_type=jnp.float32) + b_ref[:]


def _tail(aggp, n_atoms, lin2_w, lin2_b, lin_w, lin_b):
    nat = jnp.asarray(n_atoms, jnp.int32).reshape(1)
    return pl.pallas_call(
        _tail_body,
        grid=(N_NODES // RB,),
        in_specs=[
            pl.BlockSpec(memory_space=pltpu.SMEM),
            pl.BlockSpec((RB, HIDDEN), lambda i: (i, 0)),
            pl.BlockSpec((RB, HIDDEN), lambda i: (i + N_NODES // RB, 0)),
            pl.BlockSpec((HIDDEN, HIDDEN), lambda i: (0, 0)),
            pl.BlockSpec((1, HIDDEN), lambda i: (0, 0)),
            pl.BlockSpec((HIDDEN, HIDDEN), lambda i: (0, 0)),
            pl.BlockSpec((1, HIDDEN), lambda i: (0, 0)),
        ],
        out_specs=pl.BlockSpec((RB, HIDDEN), lambda i: (i, 0)),
        out_shape=jax.ShapeDtypeStruct((N_NODES, HIDDEN), jnp.float32),
    )(nat, aggp, aggp, lin2_w.T, lin2_b.reshape(1, HIDDEN), lin_w.T,
      lin_b.reshape(1, HIDDEN))


def kernel(x, edge_index, edge_weight, edge_attr, n_atoms,
           mlp_w1, mlp_b1, mlp_w2, mlp_b2,
           lin1_w, lin2_w, lin2_b, lin_w, lin_b):
    src = edge_index[1].astype(jnp.int32)
    dst = edge_index[0].astype(jnp.int32)
    xl = _proj(x, lin1_w)
    cutoff_col = _cutoff(edge_weight)
    w_edges = _filter(edge_attr, cutoff_col, mlp_w1, mlp_b1, mlp_w2, mlp_b2)
    zeros = jnp.zeros((N_NODES, HIDDEN), jnp.float32)
    aggp = _sc_conv(xl, w_edges, src, dst, zeros)
    return _tail(aggp, n_atoms, lin2_w, lin2_b, lin_w, lin_b)
